# baseline (speedup 1.0000x reference)
import jax
import jax.numpy as jnp
from jax import lax
from jax.experimental import pallas as pl
from jax.experimental.pallas import tpu as pltpu
from jax.experimental.pallas import tpu_sc as plsc

N = 10000
D = 128
E = 160000

NS = 16
C = 80
NBC = 128
EP = NS * NBC * C
NPAD = 10128
RPT = 624


def _agg_body(x_hbm, src_hbm, dst_hbm, m_hbm, deg_hbm,
              acc, degsp, rows, sidx, didx, ones, zdeg,
              gsem, ssem, dsem, fsem):
    cid = lax.axis_index("c")
    sid = lax.axis_index("s")
    rows0 = rows[0]

    def _fill_row(i, _):
        for j in range(D // 16):
            rows0[i, pl.ds(j * 16, 16)] = jnp.zeros((16,), jnp.float32)
        return 0
    lax.fori_loop(0, C, _fill_row, 0)

    def _fill_ones(i, _):
        ones[pl.ds(i * 16, 16)] = jnp.ones((16,), jnp.float32)
        return 0
    lax.fori_loop(0, C // 16, _fill_ones, 0)

    def _fill_zdeg(i, _):
        zdeg[pl.ds(i * 16, 16)] = jnp.zeros((16,), jnp.float32)
        return 0
    lax.fori_loop(0, RPT // 16, _fill_zdeg, 0)

    rbase = sid * RPT
    for k in range(RPT // C):
        pltpu.sync_copy(rows0, acc.at[pl.ds(rbase + k * C, C), :])
    rem = RPT % C
    pltpu.sync_copy(rows0.at[pl.ds(0, rem), :],
                    acc.at[pl.ds(rbase + (RPT // C) * C, rem), :])
    pltpu.sync_copy(zdeg, degsp.at[pl.ds(rbase, RPT)])

    @pl.when(sid == NS - 1)
    def _():
        tail = NPAD - NS * RPT
        pltpu.sync_copy(rows0, acc.at[pl.ds(NS * RPT, C), :])
        pltpu.sync_copy(rows0.at[pl.ds(0, tail - C), :],
                        acc.at[pl.ds(NS * RPT + C, tail - C), :])
        pltpu.sync_copy(zdeg.at[pl.ds(0, tail)],
                        degsp.at[pl.ds(NS * RPT, tail)])

    plsc.subcore_barrier()

    ebase = cid * EP + sid * (NBC * C)

    def _issue_fetch(c, j8, j4):
        off = ebase + c * C
        pltpu.async_copy(src_hbm.at[pl.ds(off, C)], sidx[j8], fsem[j4])
        pltpu.async_copy(dst_hbm.at[pl.ds(off, C)], didx[j8], fsem[j4])

    def _wait_fetch(j8, j4):
        pltpu.make_async_copy(src_hbm.at[pl.ds(ebase, C)], sidx[j8],
                              fsem[j4]).wait()
        pltpu.make_async_copy(dst_hbm.at[pl.ds(ebase, C)], didx[j8],
                              fsem[j4]).wait()

    for c in range(4):
        _issue_fetch(c, c, c)
    for c in range(2):
        _wait_fetch(c, c)
        pltpu.async_copy(x_hbm.at[sidx[c]], rows[c], gsem[c])

    def _blk(oi, _):
        for b in range(8):
            c = 8 * oi + b
            j4 = b % 4
            pltpu.make_async_copy(x_hbm.at[sidx[b]], rows[j4],
                                  gsem[j4]).wait()
            pltpu.async_copy(rows[j4], acc.at[didx[b]], ssem[j4], add=True)
            pltpu.async_copy(ones, degsp.at[didx[b]], dsem[j4], add=True)

            @pl.when(c + 4 < NBC)
            def _():
                _issue_fetch(c + 4, (b + 4) % 8, j4)

            k4 = (b + 2) % 4
            k8 = (b + 2) % 8

            @pl.when(c + 2 < NBC)
            def _():
                @pl.when(c >= 2)
                def _():
                    pltpu.make_async_copy(rows[k4], acc.at[didx[k8]],
                                          ssem[k4]).wait()
                    pltpu.make_async_copy(ones, degsp.at[didx[k8]],
                                          dsem[k4]).wait()
                _wait_fetch(k8, k4)
                pltpu.async_copy(x_hbm.at[sidx[k8]], rows[k4], gsem[k4])
        return 0
    lax.fori_loop(0, NBC // 8, _blk, 0)

    for j4 in range(4):
        pltpu.make_async_copy(rows[j4], acc.at[didx[j4]], ssem[j4]).wait()
        pltpu.make_async_copy(ones, degsp.at[didx[j4]], dsem[j4]).wait()

    plsc.subcore_barrier()

    pltpu.sync_copy(acc.at[pl.ds(rbase, RPT), :],
                    m_hbm.at[cid, pl.ds(rbase, RPT), :])
    pltpu.sync_copy(degsp.at[pl.ds(rbase, RPT)], zdeg)
    pltpu.sync_copy(zdeg, deg_hbm.at[pl.ds(cid * N + rbase, RPT)])

    @pl.when(sid == NS - 1)
    def _():
        tail = N - NS * RPT
        pltpu.sync_copy(acc.at[pl.ds(NS * RPT, tail), :],
                        m_hbm.at[cid, pl.ds(NS * RPT, tail), :])
        pltpu.sync_copy(degsp.at[pl.ds(NS * RPT, tail)],
                        zdeg.at[pl.ds(0, tail)])
        pltpu.sync_copy(zdeg.at[pl.ds(0, tail)],
                        deg_hbm.at[pl.ds(cid * N + NS * RPT, tail)])


_agg = pl.kernel(
    _agg_body,
    out_type=(
        jax.ShapeDtypeStruct((2, N, D), jnp.float32),
        jax.ShapeDtypeStruct((2 * N,), jnp.float32),
    ),
    mesh=plsc.VectorSubcoreMesh(core_axis_name="c", subcore_axis_name="s"),
    scratch_types=(
        pltpu.VMEM_SHARED((NPAD, D), jnp.float32),
        pltpu.VMEM_SHARED((NPAD,), jnp.float32),
        [pltpu.VMEM((C, D), jnp.float32) for _ in range(4)],
        [pltpu.VMEM((C,), jnp.int32) for _ in range(8)],
        [pltpu.VMEM((C,), jnp.int32) for _ in range(8)],
        pltpu.VMEM((C,), jnp.float32),
        pltpu.VMEM((RPT,), jnp.float32),
        [pltpu.SemaphoreType.DMA for _ in range(4)],
        [pltpu.SemaphoreType.DMA for _ in range(4)],
        [pltpu.SemaphoreType.DMA for _ in range(4)],
        [pltpu.SemaphoreType.DMA for _ in range(4)],
    ),
)

BR = 1000


def _dense0_body(m0, m1, d0, d1, w0, w1, b0, b1, o):
    a0 = m0[...] * (1.0 / jnp.maximum(d0[...], 1.0))
    a1 = m1[...] * (1.0 / jnp.maximum(d1[...], 1.0))
    h = (jnp.dot(a0, w0[...], preferred_element_type=jnp.float32)
         + jnp.dot(a1, w1[...], preferred_element_type=jnp.float32))
    o[...] = jnp.maximum(h + b0[...] + b1[...], 0.0)


def _dense1_body(m0, m1, d0, d1, h0, w0, w1, ws, b0, b1, bs, o):
    a0 = m0[...] * (1.0 / jnp.maximum(d0[...], 1.0))
    a1 = m1[...] * (1.0 / jnp.maximum(d1[...], 1.0))
    h = (jnp.dot(a0, w0[...], preferred_element_type=jnp.float32)
         + jnp.dot(a1, w1[...], preferred_element_type=jnp.float32)
         + jnp.dot(h0[...], ws[...], preferred_element_type=jnp.float32))
    o[...] = h + b0[...] + b1[...] + bs[...]


def _row_spec():
    return pl.BlockSpec((BR, D), lambda i: (i, 0))


def _deg_spec():
    return pl.BlockSpec((BR, 1), lambda i: (i, 0))


def _mat_spec():
    return pl.BlockSpec((D, D), lambda i: (0, 0))


def _bias_spec():
    return pl.BlockSpec((1, D), lambda i: (0, 0))


_dense0 = pl.pallas_call(
    _dense0_body,
    grid=(N // BR,),
    in_specs=[_row_spec(), _row_spec(), _deg_spec(), _deg_spec(),
              _mat_spec(), _mat_spec(), _bias_spec(), _bias_spec()],
    out_specs=_row_spec(),
    out_shape=jax.ShapeDtypeStruct((N, D), jnp.float32),
)

_dense1 = pl.pallas_call(
    _dense1_body,
    grid=(N // BR,),
    in_specs=[_row_spec(), _row_spec(), _deg_spec(), _deg_spec(),
              _row_spec(), _mat_spec(), _mat_spec(), _mat_spec(),
              _bias_spec(), _bias_spec(), _bias_spec()],
    out_specs=_row_spec(),
    out_shape=jax.ShapeDtypeStruct((N, D), jnp.float32),
)


@jax.jit
def kernel(x, src_l0_e0, dst_l0_e0, src_l0_e1, dst_l0_e1,
           src_l1_e0, dst_l1_e0, src_l1_e1, dst_l1_e1,
           W_l0_e0, b_l0_e0, W_l0_e1, b_l0_e1,
           W_l1_e0, b_l1_e0, W_l1_e1, b_l1_e1,
           W_self, b_self):
    pad_s = jnp.zeros((EP - E,), jnp.int32)
    pad_d = N + (jnp.arange(EP - E, dtype=jnp.int32) % 128)
    src0 = jnp.concatenate([src_l0_e0, pad_s, src_l0_e1, pad_s])
    dst0 = jnp.concatenate([dst_l0_e0, pad_d, dst_l0_e1, pad_d])
    src1 = jnp.concatenate([src_l1_e0, pad_s, src_l1_e1, pad_s])
    dst1 = jnp.concatenate([dst_l1_e0, pad_d, dst_l1_e1, pad_d])

    m0, deg0 = _agg(x, src0, dst0)
    h0 = _dense0(m0[0], m0[1],
                 deg0[:N].reshape(N, 1), deg0[N:].reshape(N, 1),
                 W_l0_e0, W_l0_e1,
                 b_l0_e0.reshape(1, D), b_l0_e1.reshape(1, D))

    m1, deg1 = _agg(h0, src1, dst1)
    out = _dense1(m1[0], m1[1],
                  deg1[:N].reshape(N, 1), deg1[N:].reshape(N, 1),
                  h0, W_l1_e0, W_l1_e1, W_self,
                  b_l1_e0.reshape(1, D), b_l1_e1.reshape(1, D),
                  b_self.reshape(1, D))
    return out


# speedup vs baseline: 1.5177x; 1.5177x over previous
import jax
import jax.numpy as jnp
from jax import lax
from jax.experimental import pallas as pl
from jax.experimental.pallas import tpu as pltpu
from jax.experimental.pallas import tpu_sc as plsc

N = 10000
D = 128
E = 160000

NS = 16
C = 80
NBC = 126
EP = NS * NBC * C
NPAD = 10128
RPT = 624


def _agg_body(x_hbm, sd_hbm, m_hbm, deg_hbm,
              acc, degsp, rows0, rows1, sd0, sd1, ones, zdeg,
              gsem0, gsem1):
    cid = lax.axis_index("c")
    sid = lax.axis_index("s")

    def _fill_row(i, _):
        for j in range(D // 16):
            rows0[i, pl.ds(j * 16, 16)] = jnp.zeros((16,), jnp.float32)
        return 0
    lax.fori_loop(0, C, _fill_row, 0)

    def _fill_ones(i, _):
        ones[pl.ds(i * 16, 16)] = jnp.ones((16,), jnp.float32)
        return 0
    lax.fori_loop(0, C // 16, _fill_ones, 0)

    def _fill_zdeg(i, _):
        zdeg[pl.ds(i * 16, 16)] = jnp.zeros((16,), jnp.float32)
        return 0
    lax.fori_loop(0, RPT // 16, _fill_zdeg, 0)

    rbase = sid * RPT
    for k in range(RPT // C):
        pltpu.sync_copy(rows0, acc.at[pl.ds(rbase + k * C, C), :])
    rem = RPT % C
    pltpu.sync_copy(rows0.at[pl.ds(0, rem), :],
                    acc.at[pl.ds(rbase + (RPT // C) * C, rem), :])
    pltpu.sync_copy(zdeg, degsp.at[pl.ds(rbase, RPT)])

    @pl.when(sid == NS - 1)
    def _():
        tail = NPAD - NS * RPT
        pltpu.sync_copy(rows0, acc.at[pl.ds(NS * RPT, C), :])
        pltpu.sync_copy(rows0.at[pl.ds(0, tail - C), :],
                        acc.at[pl.ds(NS * RPT + C, tail - C), :])
        pltpu.sync_copy(zdeg.at[pl.ds(0, tail)],
                        degsp.at[pl.ds(NS * RPT, tail)])

    plsc.subcore_barrier()

    cbase = cid * (EP // C) + sid * NBC

    _fetch0 = lambda c, sd: pltpu.sync_copy(sd_hbm.at[cbase + c], sd)

    _fetch0(0, sd0)
    pltpu.async_copy(x_hbm.at[sd0.at[0]], rows0, gsem0)

    def _blk(i, _):
        c0 = 2 * i
        _fetch0(c0 + 1, sd1)
        pltpu.async_copy(x_hbm.at[sd1.at[0]], rows1, gsem1)
        pltpu.make_async_copy(x_hbm.at[sd0.at[0]], rows0, gsem0).wait()
        pltpu.sync_copy(rows0, acc.at[sd0.at[1]], add=True)
        pltpu.sync_copy(ones, degsp.at[sd0.at[1]], add=True)

        @pl.when(i < NBC // 2 - 1)
        def _():
            _fetch0(c0 + 2, sd0)
            pltpu.async_copy(x_hbm.at[sd0.at[0]], rows0, gsem0)

        pltpu.make_async_copy(x_hbm.at[sd1.at[0]], rows1, gsem1).wait()
        pltpu.sync_copy(rows1, acc.at[sd1.at[1]], add=True)
        pltpu.sync_copy(ones, degsp.at[sd1.at[1]], add=True)
        return 0
    lax.fori_loop(0, NBC // 2, _blk, 0)

    plsc.subcore_barrier()

    pltpu.sync_copy(acc.at[pl.ds(rbase, RPT), :],
                    m_hbm.at[cid, pl.ds(rbase, RPT), :])
    pltpu.sync_copy(degsp.at[pl.ds(rbase, RPT)], zdeg)
    pltpu.sync_copy(zdeg, deg_hbm.at[pl.ds(cid * N + rbase, RPT)])

    @pl.when(sid == NS - 1)
    def _():
        tail = N - NS * RPT
        pltpu.sync_copy(acc.at[pl.ds(NS * RPT, tail), :],
                        m_hbm.at[cid, pl.ds(NS * RPT, tail), :])
        pltpu.sync_copy(degsp.at[pl.ds(NS * RPT, tail)],
                        zdeg.at[pl.ds(0, tail)])
        pltpu.sync_copy(zdeg.at[pl.ds(0, tail)],
                        deg_hbm.at[pl.ds(cid * N + NS * RPT, tail)])


_agg = pl.kernel(
    _agg_body,
    out_type=(
        jax.ShapeDtypeStruct((2, N, D), jnp.float32),
        jax.ShapeDtypeStruct((2 * N,), jnp.float32),
    ),
    mesh=plsc.VectorSubcoreMesh(core_axis_name="c", subcore_axis_name="s"),
    scratch_types=(
        pltpu.VMEM_SHARED((NPAD, D), jnp.float32),
        pltpu.VMEM_SHARED((NPAD,), jnp.float32),
        pltpu.VMEM((C, D), jnp.float32),
        pltpu.VMEM((C, D), jnp.float32),
        pltpu.VMEM((2, C), jnp.int32),
        pltpu.VMEM((2, C), jnp.int32),
        pltpu.VMEM((C,), jnp.float32),
        pltpu.VMEM((RPT,), jnp.float32),
        pltpu.SemaphoreType.DMA,
        pltpu.SemaphoreType.DMA,
    ),
)

BR = 1000


def _dense0_body(m0, m1, d0, d1, w0, w1, b0, b1, o):
    a0 = m0[...] * (1.0 / jnp.maximum(d0[...], 1.0))
    a1 = m1[...] * (1.0 / jnp.maximum(d1[...], 1.0))
    h = (jnp.dot(a0, w0[...], preferred_element_type=jnp.float32)
         + jnp.dot(a1, w1[...], preferred_element_type=jnp.float32))
    o[...] = jnp.maximum(h + b0[...] + b1[...], 0.0)


def _dense1_body(m0, m1, d0, d1, h0, w0, w1, ws, b0, b1, bs, o):
    a0 = m0[...] * (1.0 / jnp.maximum(d0[...], 1.0))
    a1 = m1[...] * (1.0 / jnp.maximum(d1[...], 1.0))
    h = (jnp.dot(a0, w0[...], preferred_element_type=jnp.float32)
         + jnp.dot(a1, w1[...], preferred_element_type=jnp.float32)
         + jnp.dot(h0[...], ws[...], preferred_element_type=jnp.float32))
    o[...] = h + b0[...] + b1[...] + bs[...]


def _row_spec():
    return pl.BlockSpec((BR, D), lambda i: (i, 0))


def _deg_spec():
    return pl.BlockSpec((BR, 1), lambda i: (i, 0))


def _mat_spec():
    return pl.BlockSpec((D, D), lambda i: (0, 0))


def _bias_spec():
    return pl.BlockSpec((1, D), lambda i: (0, 0))


_dense0 = pl.pallas_call(
    _dense0_body,
    grid=(N // BR,),
    in_specs=[_row_spec(), _row_spec(), _deg_spec(), _deg_spec(),
              _mat_spec(), _mat_spec(), _bias_spec(), _bias_spec()],
    out_specs=_row_spec(),
    out_shape=jax.ShapeDtypeStruct((N, D), jnp.float32),
)

_dense1 = pl.pallas_call(
    _dense1_body,
    grid=(N // BR,),
    in_specs=[_row_spec(), _row_spec(), _deg_spec(), _deg_spec(),
              _row_spec(), _mat_spec(), _mat_spec(), _mat_spec(),
              _bias_spec(), _bias_spec(), _bias_spec()],
    out_specs=_row_spec(),
    out_shape=jax.ShapeDtypeStruct((N, D), jnp.float32),
)


@jax.jit
def kernel(x, src_l0_e0, dst_l0_e0, src_l0_e1, dst_l0_e1,
           src_l1_e0, dst_l1_e0, src_l1_e1, dst_l1_e1,
           W_l0_e0, b_l0_e0, W_l0_e1, b_l0_e1,
           W_l1_e0, b_l1_e0, W_l1_e1, b_l1_e1,
           W_self, b_self):
    pad_s = jnp.zeros((EP - E,), jnp.int32)
    pad_d = N + (jnp.arange(EP - E, dtype=jnp.int32) % 128)

    def _mk_sd(s, d):
        sp = jnp.concatenate([s, pad_s]).reshape(-1, 1, C)
        dp = jnp.concatenate([d, pad_d]).reshape(-1, 1, C)
        return jnp.concatenate([sp, dp], axis=1)

    sd_a = jnp.concatenate([_mk_sd(src_l0_e0, dst_l0_e0),
                            _mk_sd(src_l0_e1, dst_l0_e1)], axis=0)
    sd_b = jnp.concatenate([_mk_sd(src_l1_e0, dst_l1_e0),
                            _mk_sd(src_l1_e1, dst_l1_e1)], axis=0)

    m0, deg0 = _agg(x, sd_a)
    h0 = _dense0(m0[0], m0[1],
                 deg0[:N].reshape(N, 1), deg0[N:].reshape(N, 1),
                 W_l0_e0, W_l0_e1,
                 b_l0_e0.reshape(1, D), b_l0_e1.reshape(1, D))

    m1, deg1 = _agg(h0, sd_b)
    out = _dense1(m1[0], m1[1],
                  deg1[:N].reshape(N, 1), deg1[N:].reshape(N, 1),
                  h0, W_l1_e0, W_l1_e1, W_self,
                  b_l1_e0.reshape(1, D), b_l1_e1.reshape(1, D),
                  b_self.reshape(1, D))
    return out


# speedup vs baseline: 1.5340x; 1.0108x over previous
import jax
import jax.numpy as jnp
from jax import lax
from jax.experimental import pallas as pl
from jax.experimental.pallas import tpu as pltpu
from jax.experimental.pallas import tpu_sc as plsc

N = 10000
D = 128
E = 160000

NS = 16
C = 80
NBC = 126
EP = NS * NBC * C
NPAD = 10128
RPT = 624


def _agg_body(x_hbm, sd_hbm, m_hbm, deg_hbm,
              acc, degsp, rows0, rows1, sd0, sd1, ones, zdeg,
              gsem0, gsem1, dsem0, dsem1):
    cid = lax.axis_index("c")
    sid = lax.axis_index("s")

    def _fill_row(i, _):
        for j in range(D // 16):
            rows0[i, pl.ds(j * 16, 16)] = jnp.zeros((16,), jnp.float32)
        return 0
    lax.fori_loop(0, C, _fill_row, 0)

    def _fill_ones(i, _):
        ones[pl.ds(i * 16, 16)] = jnp.ones((16,), jnp.float32)
        return 0
    lax.fori_loop(0, C // 16, _fill_ones, 0)

    def _fill_zdeg(i, _):
        zdeg[pl.ds(i * 16, 16)] = jnp.zeros((16,), jnp.float32)
        return 0
    lax.fori_loop(0, RPT // 16, _fill_zdeg, 0)

    rbase = sid * RPT
    for k in range(RPT // C):
        pltpu.sync_copy(rows0, acc.at[pl.ds(rbase + k * C, C), :])
    rem = RPT % C
    pltpu.sync_copy(rows0.at[pl.ds(0, rem), :],
                    acc.at[pl.ds(rbase + (RPT // C) * C, rem), :])
    pltpu.sync_copy(zdeg, degsp.at[pl.ds(rbase, RPT)])

    @pl.when(sid == NS - 1)
    def _():
        tail = NPAD - NS * RPT
        pltpu.sync_copy(rows0, acc.at[pl.ds(NS * RPT, C), :])
        pltpu.sync_copy(rows0.at[pl.ds(0, tail - C), :],
                        acc.at[pl.ds(NS * RPT + C, tail - C), :])
        pltpu.sync_copy(zdeg.at[pl.ds(0, tail)],
                        degsp.at[pl.ds(NS * RPT, tail)])

    plsc.subcore_barrier()

    cbase = cid * (EP // C) + sid * NBC

    _fetch0 = lambda c, sd: pltpu.sync_copy(sd_hbm.at[cbase + c], sd)

    _fetch0(0, sd0)
    pltpu.async_copy(x_hbm.at[sd0.at[0]], rows0, gsem0)

    def _blk(i, _):
        c0 = 2 * i
        _fetch0(c0 + 1, sd1)
        pltpu.async_copy(x_hbm.at[sd1.at[0]], rows1, gsem1)
        pltpu.make_async_copy(x_hbm.at[sd0.at[0]], rows0, gsem0).wait()
        pltpu.async_copy(ones, degsp.at[sd0.at[1]], dsem0, add=True)
        pltpu.sync_copy(rows0, acc.at[sd0.at[1]], add=True)
        pltpu.make_async_copy(ones, degsp.at[sd0.at[1]], dsem0).wait()

        @pl.when(i < NBC // 2 - 1)
        def _():
            _fetch0(c0 + 2, sd0)
            pltpu.async_copy(x_hbm.at[sd0.at[0]], rows0, gsem0)

        pltpu.make_async_copy(x_hbm.at[sd1.at[0]], rows1, gsem1).wait()
        pltpu.async_copy(ones, degsp.at[sd1.at[1]], dsem1, add=True)
        pltpu.sync_copy(rows1, acc.at[sd1.at[1]], add=True)
        pltpu.make_async_copy(ones, degsp.at[sd1.at[1]], dsem1).wait()
        return 0
    lax.fori_loop(0, NBC // 2, _blk, 0)

    plsc.subcore_barrier()

    pltpu.sync_copy(acc.at[pl.ds(rbase, RPT), :],
                    m_hbm.at[cid, pl.ds(rbase, RPT), :])
    pltpu.sync_copy(degsp.at[pl.ds(rbase, RPT)], zdeg)
    pltpu.sync_copy(zdeg, deg_hbm.at[pl.ds(cid * N + rbase, RPT)])

    @pl.when(sid == NS - 1)
    def _():
        tail = N - NS * RPT
        pltpu.sync_copy(acc.at[pl.ds(NS * RPT, tail), :],
                        m_hbm.at[cid, pl.ds(NS * RPT, tail), :])
        pltpu.sync_copy(degsp.at[pl.ds(NS * RPT, tail)],
                        zdeg.at[pl.ds(0, tail)])
        pltpu.sync_copy(zdeg.at[pl.ds(0, tail)],
                        deg_hbm.at[pl.ds(cid * N + NS * RPT, tail)])


_agg = pl.kernel(
    _agg_body,
    out_type=(
        jax.ShapeDtypeStruct((2, N, D), jnp.float32),
        jax.ShapeDtypeStruct((2 * N,), jnp.float32),
    ),
    mesh=plsc.VectorSubcoreMesh(core_axis_name="c", subcore_axis_name="s"),
    scratch_types=(
        pltpu.VMEM_SHARED((NPAD, D), jnp.float32),
        pltpu.VMEM_SHARED((NPAD,), jnp.float32),
        pltpu.VMEM((C, D), jnp.float32),
        pltpu.VMEM((C, D), jnp.float32),
        pltpu.VMEM((2, C), jnp.int32),
        pltpu.VMEM((2, C), jnp.int32),
        pltpu.VMEM((C,), jnp.float32),
        pltpu.VMEM((RPT,), jnp.float32),
        pltpu.SemaphoreType.DMA,
        pltpu.SemaphoreType.DMA,
        pltpu.SemaphoreType.DMA,
        pltpu.SemaphoreType.DMA,
    ),
)

BR = 1000


def _dense0_body(m0, m1, d0, d1, w0, w1, b0, b1, o):
    a0 = m0[...] * (1.0 / jnp.maximum(d0[...], 1.0))
    a1 = m1[...] * (1.0 / jnp.maximum(d1[...], 1.0))
    h = (jnp.dot(a0, w0[...], preferred_element_type=jnp.float32)
         + jnp.dot(a1, w1[...], preferred_element_type=jnp.float32))
    o[...] = jnp.maximum(h + b0[...] + b1[...], 0.0)


def _dense1_body(m0, m1, d0, d1, h0, w0, w1, ws, b0, b1, bs, o):
    a0 = m0[...] * (1.0 / jnp.maximum(d0[...], 1.0))
    a1 = m1[...] * (1.0 / jnp.maximum(d1[...], 1.0))
    h = (jnp.dot(a0, w0[...], preferred_element_type=jnp.float32)
         + jnp.dot(a1, w1[...], preferred_element_type=jnp.float32)
         + jnp.dot(h0[...], ws[...], preferred_element_type=jnp.float32))
    o[...] = h + b0[...] + b1[...] + bs[...]


def _row_spec():
    return pl.BlockSpec((BR, D), lambda i: (i, 0))


def _deg_spec():
    return pl.BlockSpec((BR, 1), lambda i: (i, 0))


def _mat_spec():
    return pl.BlockSpec((D, D), lambda i: (0, 0))


def _bias_spec():
    return pl.BlockSpec((1, D), lambda i: (0, 0))


_dense0 = pl.pallas_call(
    _dense0_body,
    grid=(N // BR,),
    in_specs=[_row_spec(), _row_spec(), _deg_spec(), _deg_spec(),
              _mat_spec(), _mat_spec(), _bias_spec(), _bias_spec()],
    out_specs=_row_spec(),
    out_shape=jax.ShapeDtypeStruct((N, D), jnp.float32),
)

_dense1 = pl.pallas_call(
    _dense1_body,
    grid=(N // BR,),
    in_specs=[_row_spec(), _row_spec(), _deg_spec(), _deg_spec(),
              _row_spec(), _mat_spec(), _mat_spec(), _mat_spec(),
              _bias_spec(), _bias_spec(), _bias_spec()],
    out_specs=_row_spec(),
    out_shape=jax.ShapeDtypeStruct((N, D), jnp.float32),
)


@jax.jit
def kernel(x, src_l0_e0, dst_l0_e0, src_l0_e1, dst_l0_e1,
           src_l1_e0, dst_l1_e0, src_l1_e1, dst_l1_e1,
           W_l0_e0, b_l0_e0, W_l0_e1, b_l0_e1,
           W_l1_e0, b_l1_e0, W_l1_e1, b_l1_e1,
           W_self, b_self):
    pad_s = jnp.zeros((EP - E,), jnp.int32)
    pad_d = N + (jnp.arange(EP - E, dtype=jnp.int32) % 128)

    def _mk_sd(s, d):
        sp = jnp.concatenate([s, pad_s]).reshape(-1, 1, C)
        dp = jnp.concatenate([d, pad_d]).reshape(-1, 1, C)
        return jnp.concatenate([sp, dp], axis=1)

    sd_a = jnp.concatenate([_mk_sd(src_l0_e0, dst_l0_e0),
                            _mk_sd(src_l0_e1, dst_l0_e1)], axis=0)
    sd_b = jnp.concatenate([_mk_sd(src_l1_e0, dst_l1_e0),
                            _mk_sd(src_l1_e1, dst_l1_e1)], axis=0)

    m0, deg0 = _agg(x, sd_a)
    h0 = _dense0(m0[0], m0[1],
                 deg0[:N].reshape(N, 1), deg0[N:].reshape(N, 1),
                 W_l0_e0, W_l0_e1,
                 b_l0_e0.reshape(1, D), b_l0_e1.reshape(1, D))

    m1, deg1 = _agg(h0, sd_b)
    out = _dense1(m1[0], m1[1],
                  deg1[:N].reshape(N, 1), deg1[N:].reshape(N, 1),
                  h0, W_l1_e0, W_l1_e1, W_self,
                  b_l1_e0.reshape(1, D), b_l1_e1.reshape(1, D),
                  b_self.reshape(1, D))
    return out


# speedup vs baseline: 1.9349x; 1.2613x over previous
import jax
import jax.numpy as jnp
from jax import lax
from jax.experimental import pallas as pl
from jax.experimental.pallas import tpu as pltpu
from jax.experimental.pallas import tpu_sc as plsc

N = 10000
D = 128
E = 160000

NS = 16
C = 80
EPT = E // NS
NBC = EPT // C
RPT = 624


def _agg_body(x_hbm, se0_hbm, de0_hbm, se1_hbm, de1_hbm, m_hbm, deg_hbm,
              acc, degsp, rows0, rows1, sidx0, didx0, sidx1, didx1,
              ones, zdeg, gsem0, gsem1, dsem0, dsem1):
    cid = lax.axis_index("c")
    sid = lax.axis_index("s")

    def _fill_row(i, _):
        for j in range(D // 16):
            rows0[i, pl.ds(j * 16, 16)] = jnp.zeros((16,), jnp.float32)
        return 0
    lax.fori_loop(0, C, _fill_row, 0)

    def _fill_ones(i, _):
        ones[pl.ds(i * 16, 16)] = jnp.ones((16,), jnp.float32)
        return 0
    lax.fori_loop(0, C // 16, _fill_ones, 0)

    def _fill_zdeg(i, _):
        zdeg[pl.ds(i * 16, 16)] = jnp.zeros((16,), jnp.float32)
        return 0
    lax.fori_loop(0, RPT // 16, _fill_zdeg, 0)

    rbase = sid * RPT
    for k in range(RPT // C):
        pltpu.sync_copy(rows0, acc.at[pl.ds(rbase + k * C, C), :])
    rem = RPT % C
    pltpu.sync_copy(rows0.at[pl.ds(0, rem), :],
                    acc.at[pl.ds(rbase + (RPT // C) * C, rem), :])
    pltpu.sync_copy(zdeg, degsp.at[pl.ds(rbase, RPT)])

    @pl.when(sid == NS - 1)
    def _():
        tail = N - NS * RPT
        pltpu.sync_copy(rows0.at[pl.ds(0, tail), :],
                        acc.at[pl.ds(NS * RPT, tail), :])
        pltpu.sync_copy(zdeg.at[pl.ds(0, tail)],
                        degsp.at[pl.ds(NS * RPT, tail)])

    plsc.subcore_barrier()

    ebase = sid * EPT

    def _run(src_hbm, dst_hbm):
        def _fetch(c, sbuf, dbuf):
            off = ebase + c * C
            pltpu.sync_copy(src_hbm.at[pl.ds(off, C)], sbuf)
            pltpu.sync_copy(dst_hbm.at[pl.ds(off, C)], dbuf)

        def _scat(rbuf, dbuf, dsem):
            pltpu.async_copy(ones, degsp.at[dbuf], dsem, add=True)
            pltpu.sync_copy(rbuf, acc.at[dbuf], add=True)
            pltpu.make_async_copy(ones, degsp.at[dbuf], dsem).wait()

        _fetch(0, sidx0, didx0)
        pltpu.async_copy(x_hbm.at[sidx0], rows0, gsem0)

        def _blk(i, _):
            c0 = 2 * i
            _fetch(c0 + 1, sidx1, didx1)
            pltpu.async_copy(x_hbm.at[sidx1], rows1, gsem1)
            pltpu.make_async_copy(x_hbm.at[sidx0], rows0, gsem0).wait()
            _scat(rows0, didx0, dsem0)
            _fetch(c0 + 2, sidx0, didx0)
            pltpu.async_copy(x_hbm.at[sidx0], rows0, gsem0)
            pltpu.make_async_copy(x_hbm.at[sidx1], rows1, gsem1).wait()
            _scat(rows1, didx1, dsem1)
            return 0
        lax.fori_loop(0, NBC // 2, _blk, 0)

        pltpu.make_async_copy(x_hbm.at[sidx0], rows0, gsem0).wait()
        _scat(rows0, didx0, dsem0)

    @pl.when(cid == 0)
    def _():
        _run(se0_hbm, de0_hbm)

    @pl.when(cid == 1)
    def _():
        _run(se1_hbm, de1_hbm)

    plsc.subcore_barrier()

    pltpu.sync_copy(acc.at[pl.ds(rbase, RPT), :],
                    m_hbm.at[cid, pl.ds(rbase, RPT), :])
    pltpu.sync_copy(degsp.at[pl.ds(rbase, RPT)], zdeg)
    pltpu.sync_copy(zdeg, deg_hbm.at[pl.ds(cid * N + rbase, RPT)])

    @pl.when(sid == NS - 1)
    def _():
        tail = N - NS * RPT
        pltpu.sync_copy(acc.at[pl.ds(NS * RPT, tail), :],
                        m_hbm.at[cid, pl.ds(NS * RPT, tail), :])
        pltpu.sync_copy(degsp.at[pl.ds(NS * RPT, tail)],
                        zdeg.at[pl.ds(0, tail)])
        pltpu.sync_copy(zdeg.at[pl.ds(0, tail)],
                        deg_hbm.at[pl.ds(cid * N + NS * RPT, tail)])


_agg = pl.kernel(
    _agg_body,
    out_type=(
        jax.ShapeDtypeStruct((2, N, D), jnp.float32),
        jax.ShapeDtypeStruct((2 * N,), jnp.float32),
    ),
    mesh=plsc.VectorSubcoreMesh(core_axis_name="c", subcore_axis_name="s"),
    scratch_types=(
        pltpu.VMEM_SHARED((N, D), jnp.float32),
        pltpu.VMEM_SHARED((N,), jnp.float32),
        pltpu.VMEM((C, D), jnp.float32),
        pltpu.VMEM((C, D), jnp.float32),
        pltpu.VMEM((C,), jnp.int32),
        pltpu.VMEM((C,), jnp.int32),
        pltpu.VMEM((C,), jnp.int32),
        pltpu.VMEM((C,), jnp.int32),
        pltpu.VMEM((C,), jnp.float32),
        pltpu.VMEM((RPT,), jnp.float32),
        pltpu.SemaphoreType.DMA,
        pltpu.SemaphoreType.DMA,
        pltpu.SemaphoreType.DMA,
        pltpu.SemaphoreType.DMA,
    ),
)

BR = 1000


def _dense0_body(m0, m1, d0, d1, w0, w1, b0, b1, o):
    a0 = m0[...] * (1.0 / jnp.maximum(d0[...], 1.0))
    a1 = m1[...] * (1.0 / jnp.maximum(d1[...], 1.0))
    h = (jnp.dot(a0, w0[...], preferred_element_type=jnp.float32)
         + jnp.dot(a1, w1[...], preferred_element_type=jnp.float32))
    o[...] = jnp.maximum(h + b0[...] + b1[...], 0.0)


def _dense1_body(m0, m1, d0, d1, h0, w0, w1, ws, b0, b1, bs, o):
    a0 = m0[...] * (1.0 / jnp.maximum(d0[...], 1.0))
    a1 = m1[...] * (1.0 / jnp.maximum(d1[...], 1.0))
    h = (jnp.dot(a0, w0[...], preferred_element_type=jnp.float32)
         + jnp.dot(a1, w1[...], preferred_element_type=jnp.float32)
         + jnp.dot(h0[...], ws[...], preferred_element_type=jnp.float32))
    o[...] = h + b0[...] + b1[...] + bs[...]


def _row_spec():
    return pl.BlockSpec((BR, D), lambda i: (i, 0))


def _deg_spec():
    return pl.BlockSpec((BR, 1), lambda i: (i, 0))


def _mat_spec():
    return pl.BlockSpec((D, D), lambda i: (0, 0))


def _bias_spec():
    return pl.BlockSpec((1, D), lambda i: (0, 0))


_dense0 = pl.pallas_call(
    _dense0_body,
    grid=(N // BR,),
    in_specs=[_row_spec(), _row_spec(), _deg_spec(), _deg_spec(),
              _mat_spec(), _mat_spec(), _bias_spec(), _bias_spec()],
    out_specs=_row_spec(),
    out_shape=jax.ShapeDtypeStruct((N, D), jnp.float32),
)

_dense1 = pl.pallas_call(
    _dense1_body,
    grid=(N // BR,),
    in_specs=[_row_spec(), _row_spec(), _deg_spec(), _deg_spec(),
              _row_spec(), _mat_spec(), _mat_spec(), _mat_spec(),
              _bias_spec(), _bias_spec(), _bias_spec()],
    out_specs=_row_spec(),
    out_shape=jax.ShapeDtypeStruct((N, D), jnp.float32),
)


@jax.jit
def kernel(x, src_l0_e0, dst_l0_e0, src_l0_e1, dst_l0_e1,
           src_l1_e0, dst_l1_e0, src_l1_e1, dst_l1_e1,
           W_l0_e0, b_l0_e0, W_l0_e1, b_l0_e1,
           W_l1_e0, b_l1_e0, W_l1_e1, b_l1_e1,
           W_self, b_self):
    m0, deg0 = _agg(x, src_l0_e0, dst_l0_e0, src_l0_e1, dst_l0_e1)
    h0 = _dense0(m0[0], m0[1],
                 deg0[:N].reshape(N, 1), deg0[N:].reshape(N, 1),
                 W_l0_e0, W_l0_e1,
                 b_l0_e0.reshape(1, D), b_l0_e1.reshape(1, D))

    m1, deg1 = _agg(h0, src_l1_e0, dst_l1_e0, src_l1_e1, dst_l1_e1)
    out = _dense1(m1[0], m1[1],
                  deg1[:N].reshape(N, 1), deg1[N:].reshape(N, 1),
                  h0, W_l1_e0, W_l1_e1, W_self,
                  b_l1_e0.reshape(1, D), b_l1_e1.reshape(1, D),
                  b_self.reshape(1, D))
    return out
